# baseline (speedup 1.0000x reference)
import jax
import jax.numpy as jnp
from jax import lax
from jax.experimental import pallas as pl
from jax.experimental.pallas import tpu as pltpu
from jax.experimental.pallas import tpu_sc as plsc

N_NODES = 10000
N_EDGES = 320000
D = 128
NC = 2
NS = 16
NW = NC * NS
L = 16
C = 80
EPT = N_EDGES // NW
NCHUNK = EPT // C

_SC_PARAMS = pltpu.CompilerParams(needs_layout_passes=False)
_MESH = dict(core_axis_name="c", subcore_axis_name="s")


def _mp_body(pk_h, cids_h, cemb_h, remb_h, aggp_h,
             cid_v, re_v, dsb0, dsb1, pk0, pk1, cix0, cix1, xs0, xs1,
             agg_sh, semp0, semp1, semx0, semx1, sems0, sems1):
    c = lax.axis_index("c")
    s = lax.axis_index("s")
    wid = s * NC + c
    A = pl.multiple_of(s * 625 - lax.rem(s, 8), 8)
    z16 = jnp.zeros((L,), jnp.float32)

    def zrow(i, carry):
        for j in range(D // L):
            xs0[i, pl.ds(j * L, L)] = z16
        return carry

    lax.fori_loop(0, C, zrow, 0)
    for k in range(7):
        pltpu.sync_copy(xs0, agg_sh.at[pl.ds(A + k * 80, 80)])
    pltpu.sync_copy(xs0.at[pl.ds(0, 72)], agg_sh.at[pl.ds(A + 560, 72)])
    pltpu.sync_copy(cids_h, cid_v)
    pltpu.sync_copy(remb_h, re_v)
    plsc.subcore_barrier()

    iota = lax.iota(jnp.int32, L)
    bufs = (xs0, xs1)
    pks = (pk0, pk1)
    cixs = (cix0, cix1)
    semps = (semp0, semp1)
    semxs = (semx0, semx1)
    semss = (sems0, sems1)
    dsbs = (dsb0, dsb1)

    def issue_idx(g, b):
        pltpu.async_copy(pk_h.at[wid, g], pks[b], semps[b])

    def wait_idx(g, b):
        pltpu.make_async_copy(pk_h.at[wid, g], pks[b], semps[b]).wait()

    def compose_gather(g, b):
        for k in range(C // L):
            sv = pks[b][0, pl.ds(k * L, L)]
            cixs[b][pl.ds(k * L, L)] = plsc.load_gather(cid_v, [sv])
        pltpu.async_copy(cemb_h.at[cixs[b]], bufs[b], semxs[b])

    def scatter_wait(g, b):
        pltpu.make_async_copy(bufs[b], agg_sh.at[dsbs[b]],
                              semss[b]).wait()

    def process(g, b):
        xs = bufs[b]
        pk = pks[b]
        for k in range(C // L):
            dsbs[b][pl.ds(k * L, L)] = pk[1, pl.ds(k * L, L)]
        pltpu.make_async_copy(cemb_h.at[cixs[b]], xs, semxs[b]).wait()

        @plsc.parallel_loop(0, C // L)
        def grp(k):
            kL = k * L
            rvec = pk[2, pl.ds(kL, L)]
            wvec = plsc.bitcast(pk[3, pl.ds(kL, L)], jnp.float32)
            for e in range(L):
                esp = jnp.full((L,), e, jnp.int32)
                rsp = jnp.take_along_axis(rvec, esp, axis=0)
                wsp = jnp.take_along_axis(wvec, esp, axis=0)
                i = kL + e
                for j in range(D // L):
                    sl = pl.ds(j * L, L)
                    rv = plsc.load_gather(re_v, [rsp, iota + (j * L)])
                    xs[i, sl] = jnp.maximum(xs[i, sl] + wsp * rv, 0.0)

        pltpu.async_copy(xs, agg_sh.at[dsbs[b]], semss[b], add=True)

    issue_idx(0, 0)
    wait_idx(0, 0)
    compose_gather(0, 0)
    issue_idx(1, 1)

    @pl.loop(0, NCHUNK - 1, step=2)
    def outer(g0):
        for b in range(2):
            g = g0 + b
            wait_idx(g + 1, 1 - b)

            @pl.when(g >= 1)
            def _():
                scatter_wait(g - 1, 1 - b)

            compose_gather(g + 1, 1 - b)
            process(g, b)

            @pl.when(g + 2 < NCHUNK)
            def _():
                issue_idx(g + 2, b)

    process(NCHUNK - 1, 0)
    scatter_wait(NCHUNK - 2, 1)
    scatter_wait(NCHUNK - 1, 0)
    plsc.subcore_barrier()
    for k in range(7):
        pltpu.sync_copy(agg_sh.at[pl.ds(A + k * 80, 80)], xs0)
        pltpu.sync_copy(
            xs0,
            aggp_h.at[pl.ds(pl.multiple_of(c * N_NODES + A + k * 80, 8), 80)])
    pltpu.sync_copy(agg_sh.at[pl.ds(A + 560, 72)], xs0.at[pl.ds(0, 72)])
    pltpu.sync_copy(
        xs0.at[pl.ds(0, 72)],
        aggp_h.at[pl.ds(pl.multiple_of(c * N_NODES + A + 560, 8), 72)])


_mp_call = pl.kernel(
    _mp_body,
    mesh=plsc.VectorSubcoreMesh(**_MESH),
    compiler_params=_SC_PARAMS,
    out_type=jax.ShapeDtypeStruct((NC * N_NODES, D), jnp.float32),
    scratch_types=[
        pltpu.VMEM((N_NODES,), jnp.int32),
        pltpu.VMEM((40, D), jnp.float32),
        pltpu.VMEM((C,), jnp.int32),
        pltpu.VMEM((C,), jnp.int32),
        pltpu.VMEM((4, C), jnp.int32),
        pltpu.VMEM((4, C), jnp.int32),
        pltpu.VMEM((C,), jnp.int32),
        pltpu.VMEM((C,), jnp.int32),
        pltpu.VMEM((C, D), jnp.float32),
        pltpu.VMEM((C, D), jnp.float32),
        pltpu.VMEM_SHARED((N_NODES, D), jnp.float32),
        pltpu.SemaphoreType.DMA,
        pltpu.SemaphoreType.DMA,
        pltpu.SemaphoreType.DMA,
        pltpu.SemaphoreType.DMA,
        pltpu.SemaphoreType.DMA,
        pltpu.SemaphoreType.DMA,
    ],
)


def _dense_body(a0_ref, a1_ref, wg_ref, bg_ref, wl_ref, bl_ref, rw_ref,
                xh_ref, xt_ref, rwo_ref):
    i = pl.program_id(0)
    agg = a0_ref[...] + a1_ref[...]
    x2 = jnp.maximum(
        lax.dot(agg, wg_ref[...]) + bg_ref[...], 0.0)
    wl = wl_ref[...]
    xh_ref[...] = lax.dot(x2, wl[0:D]) + bl_ref[...]
    xt_ref[...] = lax.dot(x2, wl[2 * D:3 * D])

    @pl.when(i == 0)
    def _():
        rwo_ref[...] = lax.dot(rw_ref[...], wl[D:2 * D])


def _dense_call(aggp, W_gnn, bg2, W_lin, bl2, rwin):
    blk = 1000
    n_blk = N_NODES // blk
    return pl.pallas_call(
        _dense_body,
        grid=(n_blk,),
        in_specs=[
            pl.BlockSpec((blk, D), lambda i: (i, 0)),
            pl.BlockSpec((blk, D), lambda i: (i + n_blk, 0)),
            pl.BlockSpec((D, D), lambda i: (0, 0)),
            pl.BlockSpec((1, D), lambda i: (0, 0)),
            pl.BlockSpec((3 * D, D), lambda i: (0, 0)),
            pl.BlockSpec((1, D), lambda i: (0, 0)),
            pl.BlockSpec((40, D), lambda i: (0, 0)),
        ],
        out_specs=[
            pl.BlockSpec((blk, D), lambda i: (i, 0)),
            pl.BlockSpec((blk, D), lambda i: (i, 0)),
            pl.BlockSpec((40, D), lambda i: (0, 0)),
        ],
        out_shape=[
            jax.ShapeDtypeStruct((N_NODES, D), jnp.float32),
            jax.ShapeDtypeStruct((N_NODES, D), jnp.float32),
            jax.ShapeDtypeStruct((40, D), jnp.float32),
        ],
    )(aggp, aggp, W_gnn, bg2, W_lin, bl2, rwin)


def _asm_body(head3_h, tail3_h, rel3_h, w3_h, xh2_h, xt2_h, rw_h, out_h,
              rw_v, hb, tb, relb, wb, gh0, gh1, gt0, gt1,
              sh0, sh1, st0, st1, so0, so1):
    c = lax.axis_index("c")
    s = lax.axis_index("s")
    wid = s * NC + c
    pltpu.sync_copy(rw_h, rw_v)
    pltpu.sync_copy(head3_h.at[wid], hb)
    pltpu.sync_copy(tail3_h.at[wid], tb)
    pltpu.sync_copy(rel3_h.at[wid], relb)
    pltpu.sync_copy(w3_h.at[wid], wb)
    iota = lax.iota(jnp.int32, L)
    ghs = (gh0, gh1)
    gts = (gt0, gt1)
    shs = (sh0, sh1)
    sts = (st0, st1)
    sos = (so0, so1)

    def obase(g):
        return pl.multiple_of(wid * EPT + g * C, 8)

    def out_wait(g, b):
        pltpu.make_async_copy(ghs[b], out_h.at[pl.ds(obase(g), C)],
                              sos[b]).wait()

    def issue(g, b):
        pltpu.async_copy(xh2_h.at[hb.at[g]], ghs[b], shs[b])
        pltpu.async_copy(xt2_h.at[tb.at[g]], gts[b], sts[b])

    def process(g, b):
        gh = ghs[b]
        gt = gts[b]
        pltpu.make_async_copy(xh2_h.at[hb.at[g]], gh, shs[b]).wait()
        pltpu.make_async_copy(xt2_h.at[tb.at[g]], gt, sts[b]).wait()

        @plsc.parallel_loop(0, C // L)
        def grp(k):
            kL = k * L
            rvec = relb[g, pl.ds(kL, L)]
            wvec = wb[g, pl.ds(kL, L)]
            for e in range(L):
                esp = jnp.full((L,), e, jnp.int32)
                rsp = jnp.take_along_axis(rvec, esp, axis=0)
                wsp = jnp.take_along_axis(wvec, esp, axis=0)
                i = kL + e
                for j in range(D // L):
                    sl = pl.ds(j * L, L)
                    gh[i, sl] = gh[i, sl] + gt[i, sl] + wsp * plsc.load_gather(
                        rw_v, [rsp, iota + (j * L)])

        pltpu.async_copy(gh, out_h.at[pl.ds(obase(g), C)], sos[b])

    issue(0, 0)

    @pl.loop(0, NCHUNK - 1, step=2)
    def outer(g0):
        for b in range(2):
            g = g0 + b

            @pl.when(g >= 1)
            def _():
                out_wait(g - 1, 1 - b)

            issue(g + 1, 1 - b)
            process(g, b)

    process(NCHUNK - 1, 0)
    out_wait(NCHUNK - 2, 1)
    out_wait(NCHUNK - 1, 0)

    slw = [rw_v[38, pl.ds(j * L, L)] for j in range(D // L)]

    @pl.when(wid < 25)
    def _():
        @pl.loop(0, 5)
        def schunk(k):
            rbase = pl.multiple_of(wid * 400 + k * C, 8)
            pltpu.sync_copy(xh2_h.at[pl.ds(rbase, C)], gh0)
            pltpu.sync_copy(xt2_h.at[pl.ds(rbase, C)], gt0)

            @plsc.parallel_loop(0, C, unroll=4)
            def srow(i):
                for j in range(D // L):
                    sl = pl.ds(j * L, L)
                    gh0[i, sl] = gh0[i, sl] + gt0[i, sl] + slw[j]

            pltpu.sync_copy(
                gh0,
                out_h.at[pl.ds(pl.multiple_of(N_EDGES + rbase, 8), C)])


_asm_call = pl.kernel(
    _asm_body,
    mesh=plsc.VectorSubcoreMesh(**_MESH),
    compiler_params=_SC_PARAMS,
    out_type=jax.ShapeDtypeStruct((N_EDGES + N_NODES, D), jnp.float32),
    scratch_types=[
        pltpu.VMEM((40, D), jnp.float32),
        pltpu.VMEM((NCHUNK, C), jnp.int32),
        pltpu.VMEM((NCHUNK, C), jnp.int32),
        pltpu.VMEM((NCHUNK, C), jnp.int32),
        pltpu.VMEM((NCHUNK, C), jnp.float32),
        pltpu.VMEM((C, D), jnp.float32),
        pltpu.VMEM((C, D), jnp.float32),
        pltpu.VMEM((C, D), jnp.float32),
        pltpu.VMEM((C, D), jnp.float32),
        pltpu.SemaphoreType.DMA,
        pltpu.SemaphoreType.DMA,
        pltpu.SemaphoreType.DMA,
        pltpu.SemaphoreType.DMA,
        pltpu.SemaphoreType.DMA,
        pltpu.SemaphoreType.DMA,
    ],
)


def kernel(concept_ids, edge_index, edge_attr, concept_embedding,
           relation_embedding, self_loop_embedding, W_gnn, b_gnn,
           W_lin, b_lin):
    src = edge_index[0]
    dst = edge_index[1]
    rel = edge_attr[:, 0].astype(jnp.int32)
    w = edge_attr[:, 1]
    src3 = src.reshape(NW, NCHUNK, C)
    dst3 = dst.reshape(NW, NCHUNK, C)
    rel3 = rel.reshape(NW, NCHUNK, C)
    w3 = w.reshape(NW, NCHUNK, C)
    wbits3 = jax.lax.bitcast_convert_type(w, jnp.int32).reshape(
        NW, NCHUNK, C)
    pk = jnp.stack([src3, dst3, rel3, wbits3], axis=2)
    remb_pad = jnp.concatenate(
        [relation_embedding, jnp.zeros((2, D), jnp.float32)], axis=0)
    rwin = jnp.concatenate(
        [relation_embedding, self_loop_embedding,
         jnp.zeros((1, D), jnp.float32)], axis=0)
    aggp = _mp_call(pk, concept_ids, concept_embedding, remb_pad)
    xh2, xt2, rwo = _dense_call(aggp, W_gnn, b_gnn.reshape(1, D), W_lin,
                                b_lin.reshape(1, D), rwin)
    return _asm_call(src3, dst3, rel3, w3, xh2, xt2, rwo)


# speedup vs baseline: 2.5296x; 2.5296x over previous
import jax
import jax.numpy as jnp
from jax import lax
from jax.experimental import pallas as pl
from jax.experimental.pallas import tpu as pltpu
from jax.experimental.pallas import tpu_sc as plsc

N_NODES = 10000
N_EDGES = 320000
D = 128
NC = 2
NS = 16
NW = NC * NS
L = 16
C = 80
EPT = N_EDGES // NW
NCHUNK = EPT // C

_SC_PARAMS = pltpu.CompilerParams(needs_layout_passes=False)
_MESH = dict(core_axis_name="c", subcore_axis_name="s")


def _mp_body(pk_h, cids_h, cemb_h, remb_h, aggp_h,
             cid_v, re_v, dsb0, dsb1, pk0, pk1, cix0, cix1, xs0, xs1,
             agg_sh, semp0, semp1, semx0, semx1, sems0, sems1):
    c = lax.axis_index("c")
    s = lax.axis_index("s")
    wid = s * NC + c
    A = pl.multiple_of(s * 625 - lax.rem(s, 8), 8)
    z16 = jnp.zeros((L,), jnp.float32)

    def zrow(i, carry):
        for j in range(D // L):
            xs0[i, pl.ds(j * L, L)] = z16
        return carry

    lax.fori_loop(0, C, zrow, 0)
    for k in range(7):
        pltpu.sync_copy(xs0, agg_sh.at[pl.ds(A + k * 80, 80)])
    pltpu.sync_copy(xs0.at[pl.ds(0, 72)], agg_sh.at[pl.ds(A + 560, 72)])
    pltpu.sync_copy(cids_h, cid_v)
    pltpu.sync_copy(remb_h, re_v)
    plsc.subcore_barrier()

    iota = lax.iota(jnp.int32, L)
    bufs = (xs0, xs1)
    pks = (pk0, pk1)
    cixs = (cix0, cix1)
    semps = (semp0, semp1)
    semxs = (semx0, semx1)
    semss = (sems0, sems1)
    dsbs = (dsb0, dsb1)

    def issue_idx(g, b):
        pltpu.async_copy(pk_h.at[wid, g], pks[b], semps[b])

    def wait_idx(g, b):
        pltpu.make_async_copy(pk_h.at[wid, g], pks[b], semps[b]).wait()

    def compose_gather(g, b):
        for k in range(C // L):
            sv = pks[b][0, pl.ds(k * L, L)]
            cixs[b][pl.ds(k * L, L)] = plsc.load_gather(cid_v, [sv])
        pltpu.async_copy(cemb_h.at[cixs[b]], bufs[b], semxs[b])

    def scatter_wait(g, b):
        pltpu.make_async_copy(bufs[b], agg_sh.at[dsbs[b]],
                              semss[b]).wait()

    def process(g, b):
        xs = bufs[b]
        pk = pks[b]
        for k in range(C // L):
            dsbs[b][pl.ds(k * L, L)] = pk[1, pl.ds(k * L, L)]
        pltpu.make_async_copy(cemb_h.at[cixs[b]], xs, semxs[b]).wait()

        @plsc.parallel_loop(0, C, unroll=4)
        def edge(i):
            i16 = jnp.full((L,), i, jnp.int32)
            rsp = plsc.load_gather(pk.at[2], [i16])
            wsp = plsc.bitcast(plsc.load_gather(pk.at[3], [i16]),
                               jnp.float32)
            for j in range(D // L):
                sl = pl.ds(j * L, L)
                rv = plsc.load_gather(re_v, [rsp, iota + (j * L)])
                xs[i, sl] = jnp.maximum(xs[i, sl] + wsp * rv, 0.0)

        pltpu.async_copy(xs, agg_sh.at[dsbs[b]], semss[b], add=True)

    issue_idx(0, 0)
    wait_idx(0, 0)
    compose_gather(0, 0)
    issue_idx(1, 1)

    @pl.loop(0, NCHUNK - 1, step=2)
    def outer(g0):
        for b in range(2):
            g = g0 + b
            wait_idx(g + 1, 1 - b)

            @pl.when(g >= 1)
            def _():
                scatter_wait(g - 1, 1 - b)

            compose_gather(g + 1, 1 - b)
            process(g, b)

            @pl.when(g + 2 < NCHUNK)
            def _():
                issue_idx(g + 2, b)

    process(NCHUNK - 1, 0)
    scatter_wait(NCHUNK - 2, 1)
    scatter_wait(NCHUNK - 1, 0)
    plsc.subcore_barrier()
    for k in range(7):
        pltpu.sync_copy(agg_sh.at[pl.ds(A + k * 80, 80)], xs0)
        pltpu.sync_copy(
            xs0,
            aggp_h.at[pl.ds(pl.multiple_of(c * N_NODES + A + k * 80, 8), 80)])
    pltpu.sync_copy(agg_sh.at[pl.ds(A + 560, 72)], xs0.at[pl.ds(0, 72)])
    pltpu.sync_copy(
        xs0.at[pl.ds(0, 72)],
        aggp_h.at[pl.ds(pl.multiple_of(c * N_NODES + A + 560, 8), 72)])


_mp_call = pl.kernel(
    _mp_body,
    mesh=plsc.VectorSubcoreMesh(**_MESH),
    compiler_params=_SC_PARAMS,
    out_type=jax.ShapeDtypeStruct((NC * N_NODES, D), jnp.float32),
    scratch_types=[
        pltpu.VMEM((N_NODES,), jnp.int32),
        pltpu.VMEM((40, D), jnp.float32),
        pltpu.VMEM((C,), jnp.int32),
        pltpu.VMEM((C,), jnp.int32),
        pltpu.VMEM((4, C), jnp.int32),
        pltpu.VMEM((4, C), jnp.int32),
        pltpu.VMEM((C,), jnp.int32),
        pltpu.VMEM((C,), jnp.int32),
        pltpu.VMEM((C, D), jnp.float32),
        pltpu.VMEM((C, D), jnp.float32),
        pltpu.VMEM_SHARED((N_NODES, D), jnp.float32),
        pltpu.SemaphoreType.DMA,
        pltpu.SemaphoreType.DMA,
        pltpu.SemaphoreType.DMA,
        pltpu.SemaphoreType.DMA,
        pltpu.SemaphoreType.DMA,
        pltpu.SemaphoreType.DMA,
    ],
)


def _dense_body(a0_ref, a1_ref, wg_ref, bg_ref, wl_ref, bl_ref, rw_ref,
                xh_ref, xt_ref, rwo_ref):
    i = pl.program_id(0)
    agg = a0_ref[...] + a1_ref[...]
    x2 = jnp.maximum(
        lax.dot(agg, wg_ref[...]) + bg_ref[...], 0.0)
    wl = wl_ref[...]
    xh_ref[...] = lax.dot(x2, wl[0:D]) + bl_ref[...]
    xt_ref[...] = lax.dot(x2, wl[2 * D:3 * D])

    @pl.when(i == 0)
    def _():
        rwo_ref[...] = lax.dot(rw_ref[...], wl[D:2 * D])


def _dense_call(aggp, W_gnn, bg2, W_lin, bl2, rwin):
    blk = 1000
    n_blk = N_NODES // blk
    return pl.pallas_call(
        _dense_body,
        grid=(n_blk,),
        in_specs=[
            pl.BlockSpec((blk, D), lambda i: (i, 0)),
            pl.BlockSpec((blk, D), lambda i: (i + n_blk, 0)),
            pl.BlockSpec((D, D), lambda i: (0, 0)),
            pl.BlockSpec((1, D), lambda i: (0, 0)),
            pl.BlockSpec((3 * D, D), lambda i: (0, 0)),
            pl.BlockSpec((1, D), lambda i: (0, 0)),
            pl.BlockSpec((40, D), lambda i: (0, 0)),
        ],
        out_specs=[
            pl.BlockSpec((blk, D), lambda i: (i, 0)),
            pl.BlockSpec((blk, D), lambda i: (i, 0)),
            pl.BlockSpec((40, D), lambda i: (0, 0)),
        ],
        out_shape=[
            jax.ShapeDtypeStruct((N_NODES, D), jnp.float32),
            jax.ShapeDtypeStruct((N_NODES, D), jnp.float32),
            jax.ShapeDtypeStruct((40, D), jnp.float32),
        ],
    )(aggp, aggp, W_gnn, bg2, W_lin, bl2, rwin)


def _asm_body(head3_h, tail3_h, rel3_h, w3_h, xh2_h, xt2_h, rw_h, out_h,
              rw_v, hb, tb, relb, wb, gh0, gh1, gt0, gt1,
              sh0, sh1, st0, st1, so0, so1):
    c = lax.axis_index("c")
    s = lax.axis_index("s")
    wid = s * NC + c
    pltpu.sync_copy(rw_h, rw_v)
    pltpu.sync_copy(head3_h.at[wid], hb)
    pltpu.sync_copy(tail3_h.at[wid], tb)
    pltpu.sync_copy(rel3_h.at[wid], relb)
    pltpu.sync_copy(w3_h.at[wid], wb)
    iota = lax.iota(jnp.int32, L)
    ghs = (gh0, gh1)
    gts = (gt0, gt1)
    shs = (sh0, sh1)
    sts = (st0, st1)
    sos = (so0, so1)

    def obase(g):
        return pl.multiple_of(wid * EPT + g * C, 8)

    def out_wait(g, b):
        pltpu.make_async_copy(ghs[b], out_h.at[pl.ds(obase(g), C)],
                              sos[b]).wait()

    def issue(g, b):
        pltpu.async_copy(xh2_h.at[hb.at[g]], ghs[b], shs[b])
        pltpu.async_copy(xt2_h.at[tb.at[g]], gts[b], sts[b])

    def process(g, b):
        gh = ghs[b]
        gt = gts[b]
        pltpu.make_async_copy(xh2_h.at[hb.at[g]], gh, shs[b]).wait()
        pltpu.make_async_copy(xt2_h.at[tb.at[g]], gt, sts[b]).wait()

        @plsc.parallel_loop(0, C, unroll=4)
        def edge(i):
            i16 = jnp.full((L,), i, jnp.int32)
            rsp = plsc.load_gather(relb.at[g], [i16])
            wsp = plsc.load_gather(wb.at[g], [i16])
            for j in range(D // L):
                sl = pl.ds(j * L, L)
                gh[i, sl] = gh[i, sl] + gt[i, sl] + wsp * plsc.load_gather(
                    rw_v, [rsp, iota + (j * L)])

        pltpu.async_copy(gh, out_h.at[pl.ds(obase(g), C)], sos[b])

    issue(0, 0)

    @pl.loop(0, NCHUNK - 1, step=2)
    def outer(g0):
        for b in range(2):
            g = g0 + b

            @pl.when(g >= 1)
            def _():
                out_wait(g - 1, 1 - b)

            issue(g + 1, 1 - b)
            process(g, b)

    process(NCHUNK - 1, 0)
    out_wait(NCHUNK - 2, 1)
    out_wait(NCHUNK - 1, 0)

    slw = [rw_v[38, pl.ds(j * L, L)] for j in range(D // L)]

    @pl.when(wid < 25)
    def _():
        @pl.loop(0, 5)
        def schunk(k):
            rbase = pl.multiple_of(wid * 400 + k * C, 8)
            pltpu.sync_copy(xh2_h.at[pl.ds(rbase, C)], gh0)
            pltpu.sync_copy(xt2_h.at[pl.ds(rbase, C)], gt0)

            @plsc.parallel_loop(0, C, unroll=4)
            def srow(i):
                for j in range(D // L):
                    sl = pl.ds(j * L, L)
                    gh0[i, sl] = gh0[i, sl] + gt0[i, sl] + slw[j]

            pltpu.sync_copy(
                gh0,
                out_h.at[pl.ds(pl.multiple_of(N_EDGES + rbase, 8), C)])


_asm_call = pl.kernel(
    _asm_body,
    mesh=plsc.VectorSubcoreMesh(**_MESH),
    compiler_params=_SC_PARAMS,
    out_type=jax.ShapeDtypeStruct((N_EDGES + N_NODES, D), jnp.float32),
    scratch_types=[
        pltpu.VMEM((40, D), jnp.float32),
        pltpu.VMEM((NCHUNK, C), jnp.int32),
        pltpu.VMEM((NCHUNK, C), jnp.int32),
        pltpu.VMEM((NCHUNK, C), jnp.int32),
        pltpu.VMEM((NCHUNK, C), jnp.float32),
        pltpu.VMEM((C, D), jnp.float32),
        pltpu.VMEM((C, D), jnp.float32),
        pltpu.VMEM((C, D), jnp.float32),
        pltpu.VMEM((C, D), jnp.float32),
        pltpu.SemaphoreType.DMA,
        pltpu.SemaphoreType.DMA,
        pltpu.SemaphoreType.DMA,
        pltpu.SemaphoreType.DMA,
        pltpu.SemaphoreType.DMA,
        pltpu.SemaphoreType.DMA,
    ],
)


def kernel(concept_ids, edge_index, edge_attr, concept_embedding,
           relation_embedding, self_loop_embedding, W_gnn, b_gnn,
           W_lin, b_lin):
    src = edge_index[0]
    dst = edge_index[1]
    rel = edge_attr[:, 0].astype(jnp.int32)
    w = edge_attr[:, 1]
    src3 = src.reshape(NW, NCHUNK, C)
    dst3 = dst.reshape(NW, NCHUNK, C)
    rel3 = rel.reshape(NW, NCHUNK, C)
    w3 = w.reshape(NW, NCHUNK, C)
    wbits3 = jax.lax.bitcast_convert_type(w, jnp.int32).reshape(
        NW, NCHUNK, C)
    pk = jnp.stack([src3, dst3, rel3, wbits3], axis=2)
    remb_pad = jnp.concatenate(
        [relation_embedding, jnp.zeros((2, D), jnp.float32)], axis=0)
    rwin = jnp.concatenate(
        [relation_embedding, self_loop_embedding,
         jnp.zeros((1, D), jnp.float32)], axis=0)
    aggp = _mp_call(pk, concept_ids, concept_embedding, remb_pad)
    xh2, xt2, rwo = _dense_call(aggp, W_gnn, b_gnn.reshape(1, D), W_lin,
                                b_lin.reshape(1, D), rwin)
    return _asm_call(src3, dst3, rel3, w3, xh2, xt2, rwo)


# speedup vs baseline: 2.5818x; 1.0207x over previous
import jax
import jax.numpy as jnp
from jax import lax
from jax.experimental import pallas as pl
from jax.experimental.pallas import tpu as pltpu
from jax.experimental.pallas import tpu_sc as plsc

N_NODES = 10000
N_EDGES = 320000
D = 128
NC = 2
NS = 16
NW = NC * NS
L = 16
C = 80
EPT = N_EDGES // NW
NCHUNK = EPT // C

_SC_PARAMS = pltpu.CompilerParams(needs_layout_passes=False)
_MESH = dict(core_axis_name="c", subcore_axis_name="s")


def _mp_body(pk_h, cids_h, cemb_h, remb_h, aggp_h,
             cid_v, re_v, dsb0, dsb1, pk0, pk1, cix0, cix1, xs0, xs1,
             agg_sh, semp0, semp1, semx0, semx1, sems0, sems1):
    c = lax.axis_index("c")
    s = lax.axis_index("s")
    wid = s * NC + c
    A = pl.multiple_of(s * 625 - lax.rem(s, 8), 8)
    z16 = jnp.zeros((L,), jnp.float32)

    def zrow(i, carry):
        for j in range(D // L):
            xs0[i, pl.ds(j * L, L)] = z16
        return carry

    lax.fori_loop(0, C, zrow, 0)
    for k in range(7):
        pltpu.sync_copy(xs0, agg_sh.at[pl.ds(A + k * 80, 80)])
    pltpu.sync_copy(xs0.at[pl.ds(0, 72)], agg_sh.at[pl.ds(A + 560, 72)])
    pltpu.sync_copy(cids_h, cid_v)
    pltpu.sync_copy(remb_h, re_v)
    plsc.subcore_barrier()

    iota = lax.iota(jnp.int32, L)
    bufs = (xs0, xs1)
    pks = (pk0, pk1)
    cixs = (cix0, cix1)
    semps = (semp0, semp1)
    semxs = (semx0, semx1)
    semss = (sems0, sems1)
    dsbs = (dsb0, dsb1)

    def issue_idx(g, b):
        pltpu.async_copy(pk_h.at[wid, g], pks[b], semps[b])

    def wait_idx(g, b):
        pltpu.make_async_copy(pk_h.at[wid, g], pks[b], semps[b]).wait()

    def compose_gather(g, b):
        for k in range(C // L):
            sv = pks[b][0, pl.ds(k * L, L)]
            cixs[b][pl.ds(k * L, L)] = plsc.load_gather(cid_v, [sv])
        pltpu.async_copy(cemb_h.at[cixs[b]], bufs[b], semxs[b])

    def scatter_wait(g, b):
        pltpu.make_async_copy(bufs[b], agg_sh.at[dsbs[b]],
                              semss[b]).wait()

    def process(g, b):
        xs = bufs[b]
        pk = pks[b]
        for k in range(C // L):
            dsbs[b][pl.ds(k * L, L)] = pk[1, pl.ds(k * L, L)]
        pltpu.make_async_copy(cemb_h.at[cixs[b]], xs, semxs[b]).wait()

        @plsc.parallel_loop(0, C, unroll=4)
        def edge(i):
            i16 = jnp.full((L,), i, jnp.int32)
            rsp = plsc.load_gather(pk.at[2], [i16])
            wsp = plsc.bitcast(plsc.load_gather(pk.at[3], [i16]),
                               jnp.float32)
            for j in range(D // L):
                sl = pl.ds(j * L, L)
                rv = plsc.load_gather(re_v, [rsp, iota + (j * L)])
                xs[i, sl] = jnp.maximum(xs[i, sl] + wsp * rv, 0.0)

        pltpu.async_copy(xs, agg_sh.at[dsbs[b]], semss[b], add=True)

    issue_idx(0, 0)
    wait_idx(0, 0)
    compose_gather(0, 0)
    issue_idx(1, 1)

    @pl.loop(0, NCHUNK - 1, step=2)
    def outer(g0):
        for b in range(2):
            g = g0 + b
            wait_idx(g + 1, 1 - b)

            @pl.when(g >= 1)
            def _():
                scatter_wait(g - 1, 1 - b)

            compose_gather(g + 1, 1 - b)
            process(g, b)

            @pl.when(g + 2 < NCHUNK)
            def _():
                issue_idx(g + 2, b)

    process(NCHUNK - 1, 0)
    scatter_wait(NCHUNK - 2, 1)
    scatter_wait(NCHUNK - 1, 0)
    plsc.subcore_barrier()
    for k in range(7):
        pltpu.sync_copy(agg_sh.at[pl.ds(A + k * 80, 80)], xs0)
        pltpu.sync_copy(
            xs0,
            aggp_h.at[pl.ds(pl.multiple_of(c * N_NODES + A + k * 80, 8), 80)])
    pltpu.sync_copy(agg_sh.at[pl.ds(A + 560, 72)], xs0.at[pl.ds(0, 72)])
    pltpu.sync_copy(
        xs0.at[pl.ds(0, 72)],
        aggp_h.at[pl.ds(pl.multiple_of(c * N_NODES + A + 560, 8), 72)])


_mp_call = pl.kernel(
    _mp_body,
    mesh=plsc.VectorSubcoreMesh(**_MESH),
    compiler_params=_SC_PARAMS,
    out_type=jax.ShapeDtypeStruct((NC * N_NODES, D), jnp.float32),
    scratch_types=[
        pltpu.VMEM((N_NODES,), jnp.int32),
        pltpu.VMEM((40, D), jnp.float32),
        pltpu.VMEM((C,), jnp.int32),
        pltpu.VMEM((C,), jnp.int32),
        pltpu.VMEM((4, C), jnp.int32),
        pltpu.VMEM((4, C), jnp.int32),
        pltpu.VMEM((C,), jnp.int32),
        pltpu.VMEM((C,), jnp.int32),
        pltpu.VMEM((C, D), jnp.float32),
        pltpu.VMEM((C, D), jnp.float32),
        pltpu.VMEM_SHARED((N_NODES, D), jnp.float32),
        pltpu.SemaphoreType.DMA,
        pltpu.SemaphoreType.DMA,
        pltpu.SemaphoreType.DMA,
        pltpu.SemaphoreType.DMA,
        pltpu.SemaphoreType.DMA,
        pltpu.SemaphoreType.DMA,
    ],
)


def _dense_body(a0_ref, a1_ref, wg_ref, bg_ref, wl_ref, bl_ref, rw_ref,
                xh_ref, xt_ref, rwo_ref):
    i = pl.program_id(0)
    agg = a0_ref[...] + a1_ref[...]
    x2 = jnp.maximum(
        lax.dot(agg, wg_ref[...]) + bg_ref[...], 0.0)
    wl = wl_ref[...]
    xh_ref[...] = lax.dot(x2, wl[0:D]) + bl_ref[...]
    xt_ref[...] = lax.dot(x2, wl[2 * D:3 * D])

    @pl.when(i == 0)
    def _():
        rwo_ref[...] = lax.dot(rw_ref[...], wl[D:2 * D])


def _dense_call(aggp, W_gnn, bg2, W_lin, bl2, rwin):
    blk = 1000
    n_blk = N_NODES // blk
    return pl.pallas_call(
        _dense_body,
        grid=(n_blk,),
        in_specs=[
            pl.BlockSpec((blk, D), lambda i: (i, 0)),
            pl.BlockSpec((blk, D), lambda i: (i + n_blk, 0)),
            pl.BlockSpec((D, D), lambda i: (0, 0)),
            pl.BlockSpec((1, D), lambda i: (0, 0)),
            pl.BlockSpec((3 * D, D), lambda i: (0, 0)),
            pl.BlockSpec((1, D), lambda i: (0, 0)),
            pl.BlockSpec((40, D), lambda i: (0, 0)),
        ],
        out_specs=[
            pl.BlockSpec((blk, D), lambda i: (i, 0)),
            pl.BlockSpec((blk, D), lambda i: (i, 0)),
            pl.BlockSpec((40, D), lambda i: (0, 0)),
        ],
        out_shape=[
            jax.ShapeDtypeStruct((N_NODES, D), jnp.float32),
            jax.ShapeDtypeStruct((N_NODES, D), jnp.float32),
            jax.ShapeDtypeStruct((40, D), jnp.float32),
        ],
    )(aggp, aggp, W_gnn, bg2, W_lin, bl2, rwin)


def _asm_body(head3_h, tail3_h, rw2_h, xh2_h, xt2_h, rw_h, out_h,
              rw_v, hb, tb, pkc0, pkc1, pkc2, gh0, gh1, gh2, gt0, gt1, gt2,
              sp0, sp1, sp2, sh0, sh1, sh2, st0, st1, st2, so0, so1, so2):
    c = lax.axis_index("c")
    s = lax.axis_index("s")
    wid = s * NC + c
    pltpu.sync_copy(rw_h, rw_v)
    pltpu.sync_copy(head3_h.at[wid], hb)
    pltpu.sync_copy(tail3_h.at[wid], tb)
    iota = lax.iota(jnp.int32, L)
    ghs = (gh0, gh1, gh2)
    gts = (gt0, gt1, gt2)
    pkcs = (pkc0, pkc1, pkc2)
    sps = (sp0, sp1, sp2)
    shs = (sh0, sh1, sh2)
    sts = (st0, st1, st2)
    sos = (so0, so1, so2)

    def obase(g):
        return pl.multiple_of(wid * EPT + g * C, 8)

    def out_wait(g, b):
        pltpu.make_async_copy(ghs[b], out_h.at[pl.ds(obase(g), C)],
                              sos[b]).wait()

    def issue(g, b):
        pltpu.async_copy(rw2_h.at[wid, g], pkcs[b], sps[b])
        pltpu.async_copy(xh2_h.at[hb.at[g]], ghs[b], shs[b])
        pltpu.async_copy(xt2_h.at[tb.at[g]], gts[b], sts[b])

    def process(g, b):
        gh = ghs[b]
        gt = gts[b]
        pkc = pkcs[b]
        pltpu.make_async_copy(rw2_h.at[wid, g], pkc, sps[b]).wait()
        pltpu.make_async_copy(xh2_h.at[hb.at[g]], gh, shs[b]).wait()
        pltpu.make_async_copy(xt2_h.at[tb.at[g]], gt, sts[b]).wait()

        @plsc.parallel_loop(0, C, unroll=4)
        def edge(i):
            i16 = jnp.full((L,), i, jnp.int32)
            rsp = plsc.load_gather(pkc.at[0], [i16])
            wsp = plsc.bitcast(plsc.load_gather(pkc.at[1], [i16]),
                               jnp.float32)
            for j in range(D // L):
                sl = pl.ds(j * L, L)
                gh[i, sl] = gh[i, sl] + gt[i, sl] + wsp * plsc.load_gather(
                    rw_v, [rsp, iota + (j * L)])

        pltpu.async_copy(gh, out_h.at[pl.ds(obase(g), C)], sos[b])

    issue(0, 0)
    issue(1, 1)

    @pl.loop(0, NCHUNK - 2, step=3)
    def outer(g0):
        for b3 in range(3):
            g = g0 + b3
            b = b3

            @pl.when(g >= 1)
            def _():
                out_wait(g - 1, (b + 2) % 3)

            issue(g + 2, (b + 2) % 3)
            process(g, b)

    out_wait(NCHUNK - 3, (NCHUNK - 3) % 3)
    process(NCHUNK - 2, (NCHUNK - 2) % 3)
    out_wait(NCHUNK - 2, (NCHUNK - 2) % 3)
    process(NCHUNK - 1, (NCHUNK - 1) % 3)
    out_wait(NCHUNK - 1, (NCHUNK - 1) % 3)

    slw = [rw_v[38, pl.ds(j * L, L)] for j in range(D // L)]

    @pl.when(wid < 25)
    def _():
        @pl.loop(0, 5)
        def schunk(k):
            rbase = pl.multiple_of(wid * 400 + k * C, 8)
            pltpu.sync_copy(xh2_h.at[pl.ds(rbase, C)], gh0)
            pltpu.sync_copy(xt2_h.at[pl.ds(rbase, C)], gt0)

            @plsc.parallel_loop(0, C, unroll=4)
            def srow(i):
                for j in range(D // L):
                    sl = pl.ds(j * L, L)
                    gh0[i, sl] = gh0[i, sl] + gt0[i, sl] + slw[j]

            pltpu.sync_copy(
                gh0,
                out_h.at[pl.ds(pl.multiple_of(N_EDGES + rbase, 8), C)])


_asm_call = pl.kernel(
    _asm_body,
    mesh=plsc.VectorSubcoreMesh(**_MESH),
    compiler_params=_SC_PARAMS,
    out_type=jax.ShapeDtypeStruct((N_EDGES + N_NODES, D), jnp.float32),
    scratch_types=[
        pltpu.VMEM((40, D), jnp.float32),
        pltpu.VMEM((NCHUNK, C), jnp.int32),
        pltpu.VMEM((NCHUNK, C), jnp.int32),
        pltpu.VMEM((2, C), jnp.int32),
        pltpu.VMEM((2, C), jnp.int32),
        pltpu.VMEM((2, C), jnp.int32),
        pltpu.VMEM((C, D), jnp.float32),
        pltpu.VMEM((C, D), jnp.float32),
        pltpu.VMEM((C, D), jnp.float32),
        pltpu.VMEM((C, D), jnp.float32),
        pltpu.VMEM((C, D), jnp.float32),
        pltpu.VMEM((C, D), jnp.float32),
        pltpu.SemaphoreType.DMA,
        pltpu.SemaphoreType.DMA,
        pltpu.SemaphoreType.DMA,
        pltpu.SemaphoreType.DMA,
        pltpu.SemaphoreType.DMA,
        pltpu.SemaphoreType.DMA,
        pltpu.SemaphoreType.DMA,
        pltpu.SemaphoreType.DMA,
        pltpu.SemaphoreType.DMA,
        pltpu.SemaphoreType.DMA,
        pltpu.SemaphoreType.DMA,
        pltpu.SemaphoreType.DMA,
    ],
)


def kernel(concept_ids, edge_index, edge_attr, concept_embedding,
           relation_embedding, self_loop_embedding, W_gnn, b_gnn,
           W_lin, b_lin):
    src = edge_index[0]
    dst = edge_index[1]
    rel = edge_attr[:, 0].astype(jnp.int32)
    w = edge_attr[:, 1]
    src3 = src.reshape(NW, NCHUNK, C)
    dst3 = dst.reshape(NW, NCHUNK, C)
    rel3 = rel.reshape(NW, NCHUNK, C)
    w3 = w.reshape(NW, NCHUNK, C)
    wbits3 = jax.lax.bitcast_convert_type(w, jnp.int32).reshape(
        NW, NCHUNK, C)
    pk = jnp.stack([src3, dst3, rel3, wbits3], axis=2)
    remb_pad = jnp.concatenate(
        [relation_embedding, jnp.zeros((2, D), jnp.float32)], axis=0)
    rwin = jnp.concatenate(
        [relation_embedding, self_loop_embedding,
         jnp.zeros((1, D), jnp.float32)], axis=0)
    aggp = _mp_call(pk, concept_ids, concept_embedding, remb_pad)
    xh2, xt2, rwo = _dense_call(aggp, W_gnn, b_gnn.reshape(1, D), W_lin,
                                b_lin.reshape(1, D), rwin)
    rw2 = jnp.stack([rel3, wbits3], axis=2)
    return _asm_call(src3, dst3, rw2, xh2, xt2, rwo)
